# baseline (speedup 1.0000x reference)
import functools
import math

import jax
import jax.numpy as jnp
from jax import lax
from jax.experimental import pallas as pl
from jax.experimental.pallas import tpu as pltpu
from jax.experimental.pallas import tpu_sc as plsc

D_MODEL = 1024
SCALE = math.sqrt(D_MODEL)
LANES = 16

_info = plsc.get_sparse_core_info()
NUM_CORES = _info.num_cores
NUM_SUBCORES = _info.num_subcores
NUM_WORKERS = NUM_CORES * NUM_SUBCORES


def _make_kernel(B: int):
    assert B % NUM_WORKERS == 0
    b_per_w = B // NUM_WORKERS
    CHUNK = 16
    assert b_per_w % CHUNK == 0
    n_chunks = b_per_w // CHUNK
    assert n_chunks >= 4 and n_chunks % 2 == 0

    mesh = plsc.VectorSubcoreMesh(core_axis_name="c", subcore_axis_name="s")

    @functools.partial(
        pl.kernel,
        mesh=mesh,
        out_type=jax.ShapeDtypeStruct((B, D_MODEL), jnp.float32),
        scratch_types=[
            pltpu.VMEM((b_per_w,), jnp.int32),
            pltpu.VMEM((CHUNK, D_MODEL), jnp.float32),
            pltpu.VMEM((CHUNK, D_MODEL), jnp.float32),
            pltpu.VMEM((CHUNK, D_MODEL), jnp.float32),
            pltpu.VMEM((CHUNK, D_MODEL), jnp.float32),
            pltpu.SemaphoreType.DMA,
            pltpu.SemaphoreType.DMA,
            pltpu.SemaphoreType.DMA,
            pltpu.SemaphoreType.DMA,
        ],
    )
    def emb_kernel(x_hbm, table_hbm, out_hbm, idx_v, g0, g1, w0, w1, gs0, gs1, ws0, ws1):
        gbuf = (g0, g1)
        wbuf = (w0, w1)
        gsem = (gs0, gs1)
        wsem = (ws0, ws1)

        wid = lax.axis_index("s") * NUM_CORES + lax.axis_index("c")
        base = wid * b_per_w
        pltpu.sync_copy(x_hbm.at[pl.ds(base, b_per_w)], idx_v)

        def start_gather(c, b):
            pltpu.async_copy(
                table_hbm.at[idx_v.at[pl.ds(c * CHUNK, CHUNK)]], gbuf[b], gsem[b]
            )

        def wait_gather(c, b):
            pltpu.make_async_copy(
                table_hbm.at[idx_v.at[pl.ds(c * CHUNK, CHUNK)]], gbuf[b], gsem[b]
            ).wait()

        def scale(b):
            @pl.loop(0, CHUNK)
            def _row(r):
                @pl.loop(0, D_MODEL // LANES, unroll=8)
                def _vec(k):
                    sl = pl.ds(k * LANES, LANES)
                    wbuf[b][r, sl] = gbuf[b][r, sl] * SCALE

        def start_write(c, b):
            pltpu.async_copy(
                wbuf[b], out_hbm.at[pl.ds(base + c * CHUNK, CHUNK)], wsem[b]
            )

        def wait_write(c, b):
            pltpu.make_async_copy(
                wbuf[b], out_hbm.at[pl.ds(base + c * CHUNK, CHUNK)], wsem[b]
            ).wait()

        for b in range(2):
            start_gather(b, b)

        for c in range(2):
            b = c % 2
            wait_gather(c, b)
            scale(b)
            start_write(c, b)
            start_gather(c + 2, b)

        @pl.loop(2, n_chunks - 2, step=2)
        def _main(j):
            for b in range(2):
                c = j + b
                wait_gather(c, b)
                wait_write(c - 2, b)
                scale(b)
                start_write(c, b)
                start_gather(c + 2, b)

        for cc in (n_chunks - 2, n_chunks - 1):
            b = cc % 2
            wait_gather(cc, b)
            wait_write(cc - 2, b)
            scale(b)
            start_write(cc, b)

        for cc in (n_chunks - 2, n_chunks - 1):
            wait_write(cc, cc % 2)

    return emb_kernel


@jax.jit
def kernel(x, table):
    B = x.shape[0] * x.shape[1]
    flat_idx = x.reshape(B).astype(jnp.int32)
    out = _make_kernel(B)(flat_idx, table)
    return out.reshape(x.shape[0], x.shape[1], D_MODEL)


# speedup vs baseline: 1.9295x; 1.9295x over previous
import functools
import math

import jax
import jax.numpy as jnp
from jax import lax
from jax.experimental import pallas as pl
from jax.experimental.pallas import tpu as pltpu
from jax.experimental.pallas import tpu_sc as plsc

D_MODEL = 1024
SCALE = math.sqrt(D_MODEL)
LANES = 16

_info = plsc.get_sparse_core_info()
NUM_CORES = _info.num_cores
NUM_SUBCORES = _info.num_subcores
NUM_WORKERS = NUM_CORES * NUM_SUBCORES


def _make_kernel(B: int):
    assert B % NUM_WORKERS == 0
    b_per_w = B // NUM_WORKERS
    CHUNK = 64
    assert b_per_w % CHUNK == 0
    n_chunks = b_per_w // CHUNK

    mesh = plsc.VectorSubcoreMesh(core_axis_name="c", subcore_axis_name="s")

    @functools.partial(
        pl.kernel,
        mesh=mesh,
        out_type=jax.ShapeDtypeStruct((B, D_MODEL), jnp.float32),
        scratch_types=[
            pltpu.VMEM((b_per_w,), jnp.int32),
            pltpu.VMEM((CHUNK, D_MODEL), jnp.float32),
            pltpu.SemaphoreType.DMA,
        ],
    )
    def emb_kernel(x_hbm, table_hbm, out_hbm, idx_v, buf, sem):
        wid = lax.axis_index("s") * NUM_CORES + lax.axis_index("c")
        base = wid * b_per_w
        pltpu.sync_copy(x_hbm.at[pl.ds(base, b_per_w)], idx_v)

        @pl.loop(0, n_chunks)
        def _chunk_loop(j):
            pltpu.async_copy(
                table_hbm.at[idx_v.at[pl.ds(j * CHUNK, CHUNK)]], buf, sem
            ).wait()

            @pl.loop(0, CHUNK)
            def _row_loop(r):
                @pl.loop(0, D_MODEL // LANES, unroll=8)
                def _vec_loop(k):
                    sl = pl.ds(k * LANES, LANES)
                    buf[r, sl] = buf[r, sl] * SCALE

            pltpu.sync_copy(buf, out_hbm.at[pl.ds(base + j * CHUNK, CHUNK)])

    return emb_kernel


@jax.jit
def kernel(x, table):
    B = x.shape[0] * x.shape[1]
    flat_idx = x.reshape(B).astype(jnp.int32)
    out = _make_kernel(B)(flat_idx, table)
    return out.reshape(x.shape[0], x.shape[1], D_MODEL)


# speedup vs baseline: 2.6056x; 1.3504x over previous
import functools
import math

import jax
import jax.numpy as jnp
from jax import lax
from jax.experimental import pallas as pl
from jax.experimental.pallas import tpu as pltpu
from jax.experimental.pallas import tpu_sc as plsc

D_MODEL = 1024
SCALE = math.sqrt(D_MODEL)
LANES = 16

_info = plsc.get_sparse_core_info()
NUM_CORES = _info.num_cores
NUM_SUBCORES = _info.num_subcores
NUM_WORKERS = NUM_CORES * NUM_SUBCORES


def _make_kernel(B: int):
    assert B % NUM_WORKERS == 0
    b_per_w = B // NUM_WORKERS
    CHUNK = 32
    NR = 3
    assert b_per_w % CHUNK == 0
    n_chunks = b_per_w // CHUNK
    assert n_chunks >= NR + 1

    mesh = plsc.VectorSubcoreMesh(core_axis_name="c", subcore_axis_name="s")

    @functools.partial(
        pl.kernel,
        mesh=mesh,
        out_type=jax.ShapeDtypeStruct((B, D_MODEL), jnp.float32),
        scratch_types=[
            pltpu.VMEM((b_per_w,), jnp.int32),
            pltpu.VMEM((NR * CHUNK, D_MODEL), jnp.float32),
            pltpu.SemaphoreType.DMA,
            pltpu.SemaphoreType.DMA,
        ],
    )
    def emb_kernel(x_hbm, table_hbm, out_hbm, idx_v, ring, gsem, wsem):
        wid = lax.axis_index("s") * NUM_CORES + lax.axis_index("c")
        base = wid * b_per_w
        pltpu.sync_copy(x_hbm.at[pl.ds(base, b_per_w)], idx_v)

        def region_off(c):
            return lax.rem(c, NR) * CHUNK

        def start_gather(c):
            pltpu.async_copy(
                table_hbm.at[idx_v.at[pl.ds(c * CHUNK, CHUNK)]],
                ring.at[pl.ds(region_off(c), CHUNK)],
                gsem,
            )

        def wait_gather():
            pltpu.make_async_copy(
                table_hbm.at[idx_v.at[pl.ds(0, CHUNK)]],
                ring.at[pl.ds(0, CHUNK)],
                gsem,
            ).wait()

        def start_write(c):
            pltpu.async_copy(
                ring.at[pl.ds(region_off(c), CHUNK)],
                out_hbm.at[pl.ds(base + c * CHUNK, CHUNK)],
                wsem,
            )

        def wait_write():
            pltpu.make_async_copy(
                ring.at[pl.ds(0, CHUNK)],
                out_hbm.at[pl.ds(base, CHUNK)],
                wsem,
            ).wait()

        start_gather(0)

        @pl.loop(0, n_chunks)
        def _slot(c):
            off = region_off(c)
            wait_gather()

            @pl.when(c + 1 < n_chunks)
            def _issue_next():
                @pl.when(c >= NR - 1)
                def _free_region():
                    wait_write()

                start_gather(c + 1)

            @pl.loop(0, CHUNK)
            def _row(r):
                @pl.loop(0, D_MODEL // LANES, unroll=8)
                def _vec(k):
                    sl = pl.ds(k * LANES, LANES)
                    ring[off + r, sl] = ring[off + r, sl] * SCALE

            start_write(c)

        remaining = n_chunks - max(0, (n_chunks - 1) - (NR - 1))
        for _ in range(remaining):
            wait_write()

    return emb_kernel


@jax.jit
def kernel(x, table):
    B = x.shape[0] * x.shape[1]
    flat_idx = x.reshape(B).astype(jnp.int32)
    out = _make_kernel(B)(flat_idx, table)
    return out.reshape(x.shape[0], x.shape[1], D_MODEL)


# speedup vs baseline: 2.6059x; 1.0001x over previous
import functools
import math

import jax
import jax.numpy as jnp
from jax import lax
from jax.experimental import pallas as pl
from jax.experimental.pallas import tpu as pltpu
from jax.experimental.pallas import tpu_sc as plsc

D_MODEL = 1024
SCALE = math.sqrt(D_MODEL)
LANES = 16

_info = plsc.get_sparse_core_info()
NUM_CORES = _info.num_cores
NUM_SUBCORES = _info.num_subcores
NUM_WORKERS = NUM_CORES * NUM_SUBCORES


def _make_kernel(B: int):
    assert B % NUM_WORKERS == 0
    b_per_w = B // NUM_WORKERS
    CHUNK = 32
    NR = 3
    assert b_per_w % CHUNK == 0
    n_chunks = b_per_w // CHUNK
    assert n_chunks >= NR + 1

    mesh = plsc.VectorSubcoreMesh(core_axis_name="c", subcore_axis_name="s")

    @functools.partial(
        pl.kernel,
        mesh=mesh,
        out_type=jax.ShapeDtypeStruct((B, D_MODEL), jnp.float32),
        scratch_types=[
            pltpu.VMEM((b_per_w,), jnp.int32),
            pltpu.VMEM((NR * CHUNK, D_MODEL), jnp.float32),
            pltpu.SemaphoreType.DMA,
            pltpu.SemaphoreType.DMA,
        ],
    )
    def emb_kernel(x_hbm, table_hbm, out_hbm, idx_v, ring, gsem, wsem):
        wid = lax.axis_index("s") * NUM_CORES + lax.axis_index("c")
        base = wid * b_per_w
        pltpu.sync_copy(x_hbm.at[pl.ds(base, b_per_w)], idx_v)

        def region_off(c):
            return lax.rem(c, NR) * CHUNK

        def start_gather(c):
            pltpu.async_copy(
                table_hbm.at[idx_v.at[pl.ds(c * CHUNK, CHUNK)]],
                ring.at[pl.ds(region_off(c), CHUNK)],
                gsem,
            )

        def wait_gather():
            pltpu.make_async_copy(
                table_hbm.at[idx_v.at[pl.ds(0, CHUNK)]],
                ring.at[pl.ds(0, CHUNK)],
                gsem,
            ).wait()

        def start_write(c):
            pltpu.async_copy(
                ring.at[pl.ds(region_off(c), CHUNK)],
                out_hbm.at[pl.ds(base + c * CHUNK, CHUNK)],
                wsem,
            )

        def wait_write():
            pltpu.make_async_copy(
                ring.at[pl.ds(0, CHUNK)],
                out_hbm.at[pl.ds(base, CHUNK)],
                wsem,
            ).wait()

        start_gather(0)

        @pl.loop(0, n_chunks)
        def _slot(c):
            off = region_off(c)
            wait_gather()

            @pl.when(c + 1 < n_chunks)
            def _issue_next():
                @pl.when(c >= NR - 1)
                def _free_region():
                    wait_write()

                start_gather(c + 1)

            @pl.loop(0, CHUNK)
            def _row(r):
                @plsc.parallel_loop(0, D_MODEL // LANES, unroll=8)
                def _vec(k):
                    sl = pl.ds(k * LANES, LANES)
                    ring[off + r, sl] = ring[off + r, sl] * SCALE

            start_write(c)

        remaining = n_chunks - max(0, (n_chunks - 1) - (NR - 1))
        for _ in range(remaining):
            wait_write()

    return emb_kernel


@jax.jit
def kernel(x, table):
    B = x.shape[0] * x.shape[1]
    flat_idx = x.reshape(B).astype(jnp.int32)
    out = _make_kernel(B)(flat_idx, table)
    return out.reshape(x.shape[0], x.shape[1], D_MODEL)
